# baseline (speedup 1.0000x reference)
import functools

import numpy as np
import jax
import jax.numpy as jnp
from jax import lax
from jax.experimental import pallas as pl
from jax.experimental.pallas import tpu as pltpu
from jax.experimental.pallas import tpu_sc as plsc

B, L = 4096, 200
D = 64
LANES = 16
NGRP = D // LANES

NC, NS = 2, 16
NW = NC * NS
BPW = B // NW
CHUNK = L // 2
NCHUNK = 2 * BPW
NBUF = 4
SC_ROWS = NS * BPW

_SEG = np.repeat(
    (np.arange(NS, dtype=np.int32)[:, None] * BPW
     + (np.arange(NCHUNK, dtype=np.int32) // 2)[None, :]),
    CHUNK, axis=1).reshape(NS, NCHUNK, CHUNK)


def _sc_embed_sum(x2, table, seg):
    mesh = plsc.VectorSubcoreMesh(core_axis_name="c", subcore_axis_name="s")

    @functools.partial(
        pl.kernel,
        out_type=jax.ShapeDtypeStruct((B, D), jnp.float32),
        mesh=mesh,
        compiler_params=pltpu.CompilerParams(use_tc_tiling_on_sc=False),
        scratch_types=[
            pltpu.VMEM((NCHUNK, CHUNK), jnp.int32),
            pltpu.VMEM((NCHUNK, CHUNK), jnp.int32),
            pltpu.VMEM((NBUF, CHUNK, D), jnp.float32),
            pltpu.VMEM((BPW, D), jnp.float32),
            pltpu.VMEM_SHARED((SC_ROWS, D), jnp.float32),
        ] + [pltpu.SemaphoreType.DMA] * (2 * NBUF),
    )
    def k(x_hbm, table_hbm, seg_hbm, out_hbm, idx_v, seg_v, rows_v, zero_v,
          acc, *sems):
        gsem = sems[:NBUF]
        ssem = sems[NBUF:]
        s = lax.axis_index("s")
        c = lax.axis_index("c")
        wid = s * NC + c
        base = wid * NCHUNK

        def zbody(r, _):
            for g in range(NGRP):
                zero_v[r, pl.ds(g * LANES, LANES)] = jnp.zeros(
                    (LANES,), jnp.float32)
        lax.fori_loop(0, BPW, zbody, None)
        pltpu.sync_copy(zero_v, acc.at[pl.ds(s * BPW, BPW)])

        pltpu.sync_copy(x_hbm.at[pl.ds(base, NCHUNK)], idx_v)
        pltpu.sync_copy(seg_hbm.at[s], seg_v)

        def start_gather(j, p):
            return pltpu.async_copy(
                table_hbm.at[idx_v.at[j]], rows_v.at[p], gsem[p])

        for p in range(NBUF):
            start_gather(p, p)

        def outer(jj):
            for p in range(NBUF):
                j = jj + p
                pltpu.make_async_copy(
                    table_hbm.at[idx_v.at[j]], rows_v.at[p], gsem[p]).wait()
                pltpu.async_copy(
                    rows_v.at[p], acc.at[seg_v.at[j]], ssem[p], add=True)

                @pl.when(j + NBUF < NCHUNK)
                def _():
                    pltpu.make_async_copy(
                        rows_v.at[p], acc.at[seg_v.at[j]], ssem[p]).wait()
                    start_gather(j + NBUF, p)

        pl.loop(0, NCHUNK, step=NBUF)(outer)

        for p in range(NBUF):
            j = NCHUNK - NBUF + p
            pltpu.make_async_copy(
                rows_v.at[p], acc.at[seg_v.at[j]], ssem[p]).wait()
        pltpu.sync_copy(acc.at[pl.ds(s * BPW, BPW)],
                        out_hbm.at[pl.ds(wid * BPW, BPW)])

    return k(x2, table, seg)


def _tc_mlp(h_sum, W1t, b1, W2t, b2, W3t, b3):
    def body(h_ref, w1_ref, b1_ref, w2_ref, b2_ref, w3_ref, b3_ref, o_ref):
        h = h_ref[...] * (1.0 / L)
        h = jnp.dot(h, w1_ref[...], preferred_element_type=jnp.float32)
        h = jnp.maximum(h + b1_ref[...], 0.0)
        h = jnp.dot(h, w2_ref[...], preferred_element_type=jnp.float32)
        h = jnp.maximum(h + b2_ref[...], 0.0)
        o = jnp.dot(h, w3_ref[...], preferred_element_type=jnp.float32)
        o_ref[...] = o + b3_ref[...]

    return pl.pallas_call(
        body,
        out_shape=jax.ShapeDtypeStruct((B, W3t.shape[1]), jnp.float32),
    )(h_sum, W1t, b1[None, :], W2t, b2[None, :], W3t, b3[None, :])


def kernel(x, table, W1, b1, W2, b2, W3, b3):
    x2 = x.reshape(2 * B, L // 2)
    seg = jnp.asarray(_SEG)
    h_sum = _sc_embed_sum(x2, table, seg)
    return _tc_mlp(h_sum, W1.T, b1, W2.T, b2, W3.T, b3)


# speedup vs baseline: 2.2434x; 2.2434x over previous
import functools

import jax
import jax.numpy as jnp
from jax import lax
from jax.experimental import pallas as pl
from jax.experimental.pallas import tpu as pltpu
from jax.experimental.pallas import tpu_sc as plsc

B, L = 4096, 200
V, D = 1000000, 64
LANES = 16
CW = D // 2

NC, NS = 2, 16
NW = NC * NS
BPW = B // NW
CHUNK = L // 2
NCHUNK = 2 * BPW
NBUF = 8

PBLK = 2048
GRP = 4
NPB = -(-V // (GRP * PBLK))
VP2 = NPB * PBLK
VBIG = GRP * VP2
MAXB = V // PBLK


def _tc_repack(tableT):
    def pack(blk):
        u = jax.lax.bitcast_convert_type(blk, jnp.uint32)
        lo = (u[:CW, :] + jnp.uint32(0x8000)) >> 16
        hi = (u[CW:, :] + jnp.uint32(0x8000)) & jnp.uint32(0xFFFF0000)
        packed = jax.lax.bitcast_convert_type(lo | hi, jnp.float32)
        return jnp.swapaxes(packed, 0, 1)

    def body(a_ref, b_ref, c_ref, d_ref, o_ref):
        o_ref[...] = jnp.concatenate(
            [pack(a_ref[...]), pack(b_ref[...]),
             pack(c_ref[...]), pack(d_ref[...])], axis=1)

    return pl.pallas_call(
        body,
        grid=(NPB,),
        in_specs=[
            pl.BlockSpec(
                (D, PBLK),
                (lambda i, j=j: (0, jnp.minimum(GRP * i + j, MAXB))))
            for j in range(GRP)
        ],
        out_specs=pl.BlockSpec((PBLK, 2 * D), lambda i: (i, 0)),
        out_shape=jax.ShapeDtypeStruct((VP2, 2 * D), jnp.float32),
    )(tableT, tableT, tableT, tableT)


def _sc_embed_sum(x2, tableC):
    mesh = plsc.VectorSubcoreMesh(core_axis_name="c", subcore_axis_name="s")

    @functools.partial(
        pl.kernel,
        out_type=jax.ShapeDtypeStruct((B, D), jnp.float32),
        mesh=mesh,
        compiler_params=pltpu.CompilerParams(use_tc_tiling_on_sc=False,
                                     needs_layout_passes=False),
        scratch_types=[
            pltpu.VMEM((NCHUNK, CHUNK), jnp.int32),
            pltpu.VMEM((NBUF, CHUNK, CW), jnp.float32),
            pltpu.VMEM((BPW, D), jnp.float32),
        ] + [pltpu.SemaphoreType.DMA] * NBUF,
    )
    def k(x_hbm, table_hbm, out_hbm, idx_v, rows_v, out_v, *sems):
        wid = lax.axis_index("s") * NC + lax.axis_index("c")
        base = wid * NCHUNK
        pltpu.sync_copy(x_hbm.at[pl.ds(base, NCHUNK)], idx_v)

        def start(j, p):
            return pltpu.async_copy(
                table_hbm.at[idx_v.at[j]], rows_v.at[p], sems[p])

        for p in range(NBUF):
            start(p, p)

        mask_hi = jnp.full((LANES,), 0xFFFF0000, jnp.uint32)

        def sum_chunk(p, accs):
            buf = rows_v.at[p]

            def rbody(r, a):
                out = list(a)
                for w in range(2):
                    cv = buf[r, pl.ds(w * LANES, LANES)]
                    u = plsc.bitcast(cv, jnp.uint32)
                    flo = plsc.bitcast(u << 16, jnp.float32)
                    fhi = plsc.bitcast(u & mask_hi, jnp.float32)
                    out[w] = out[w] + flo
                    out[w + 2] = out[w + 2] + fhi
                return tuple(out)

            return lax.fori_loop(0, CHUNK, rbody, accs, unroll=4)

        def outer(jj):
            for b in range(NBUF // 2):
                accs = tuple(jnp.zeros((LANES,), jnp.float32)
                             for _ in range(4))
                for h in range(2):
                    p = 2 * b + h
                    j = jj + p
                    pltpu.make_async_copy(
                        table_hbm.at[idx_v.at[j]], rows_v.at[p], sems[p]
                    ).wait()
                    accs = sum_chunk(p, accs)

                    @pl.when(j + NBUF < NCHUNK)
                    def _():
                        start(j + NBUF, p)

                row = jj // 2 + b
                for g in range(4):
                    out_v[row, pl.ds(g * LANES, LANES)] = accs[g]

        pl.loop(0, NCHUNK, step=NBUF)(outer)
        pltpu.sync_copy(out_v, out_hbm.at[pl.ds(wid * BPW, BPW)])

    return k(x2, tableC)


def _tc_mlp(h_sum, W1t, b1, W2t, b2, W3t, b3):
    def body(h_ref, w1_ref, b1_ref, w2_ref, b2_ref, w3_ref, b3_ref, o_ref):
        h = h_ref[...] * (1.0 / L)
        h = jnp.dot(h, w1_ref[...], preferred_element_type=jnp.float32)
        h = jnp.maximum(h + b1_ref[...], 0.0)
        h = jnp.dot(h, w2_ref[...], preferred_element_type=jnp.float32)
        h = jnp.maximum(h + b2_ref[...], 0.0)
        o = jnp.dot(h, w3_ref[...], preferred_element_type=jnp.float32)
        o_ref[...] = o + b3_ref[...]

    return pl.pallas_call(
        body,
        out_shape=jax.ShapeDtypeStruct((B, W3t.shape[1]), jnp.float32),
    )(h_sum, W1t, b1[None, :], W2t, b2[None, :], W3t, b3[None, :])


def kernel(x, table, W1, b1, W2, b2, W3, b3):
    t2 = _tc_repack(table.T)
    tableC = t2.reshape(VBIG, CW)
    xr = ((x // (GRP * PBLK)) * (GRP * PBLK) + (x % PBLK) * GRP
          + (x // PBLK) % GRP)
    x2 = xr.reshape(2 * B, L // 2)
    h_sum = _sc_embed_sum(x2, tableC)
    return _tc_mlp(h_sum, W1.T, b1, W2.T, b2, W3.T, b3)
